# baseline (speedup 1.0000x reference)
import jax
import jax.numpy as jnp
from jax.experimental import pallas as pl
from jax.experimental.pallas import tpu as pltpu

_MIN_MASK_AREA = 120000
_THRESHOLD = 0.5
_N, _H, _W = 128, 512, 512


def _body(masks_ref, labels_ref, boxes_ref, ids_ref,
          masks_out_ref, labels_out_ref, boxes_out_ref, ids_out_ref,
          keep0_ref, idx_ref, cnt_ref, sem):
    i = pl.program_id(0)

    @pl.when(i == 0)
    def _():
        cnt_ref[0] = 0
        keep0_ref[...] = masks_ref[...]

    area = jnp.sum((masks_ref[...] > _THRESHOLD).astype(jnp.int32))

    @pl.when(area >= _MIN_MASK_AREA)
    def _():
        c = cnt_ref[0]
        idx_ref[c] = i
        cp = pltpu.make_async_copy(masks_ref, masks_out_ref.at[pl.ds(c, 1)], sem)
        cp.start()
        cp.wait()
        cnt_ref[0] = c + 1

    @pl.when(i == _N - 1)
    def _():
        c = cnt_ref[0]

        def _fill(k, carry):
            @pl.when(k >= c)
            def _():
                idx_ref[k] = 0
                cp = pltpu.make_async_copy(
                    keep0_ref, masks_out_ref.at[pl.ds(k, 1)], sem)
                cp.start()
                cp.wait()
            return carry

        jax.lax.fori_loop(0, _N, _fill, 0)

        def _gather(j, carry):
            t = idx_ref[j]
            labels_out_ref[j] = labels_ref[t]
            ids_out_ref[j] = ids_ref[t]
            for col in range(9):
                boxes_out_ref[j, col] = boxes_ref[t, col]
            return carry

        jax.lax.fori_loop(0, _N, _gather, 0)


def kernel(masks, labels, boxes_3d, instance_ids):
    out_shape = (
        jax.ShapeDtypeStruct((_N, _H, _W), jnp.float32),
        jax.ShapeDtypeStruct((_N,), jnp.int32),
        jax.ShapeDtypeStruct((_N, 9), jnp.float32),
        jax.ShapeDtypeStruct((_N,), jnp.int32),
    )
    return pl.pallas_call(
        _body,
        grid=(_N,),
        in_specs=[
            pl.BlockSpec((1, _H, _W), lambda i: (i, 0, 0)),
            pl.BlockSpec(memory_space=pltpu.SMEM),
            pl.BlockSpec(memory_space=pltpu.SMEM),
            pl.BlockSpec(memory_space=pltpu.SMEM),
        ],
        out_specs=(
            pl.BlockSpec(memory_space=pltpu.HBM),
            pl.BlockSpec(memory_space=pltpu.SMEM),
            pl.BlockSpec(memory_space=pltpu.SMEM),
            pl.BlockSpec(memory_space=pltpu.SMEM),
        ),
        out_shape=out_shape,
        scratch_shapes=[
            pltpu.VMEM((1, _H, _W), jnp.float32),
            pltpu.SMEM((_N,), jnp.int32),
            pltpu.SMEM((1,), jnp.int32),
            pltpu.SemaphoreType.DMA,
        ],
    )(masks, labels, boxes_3d, instance_ids)


# speedup vs baseline: 1.2916x; 1.2916x over previous
import jax
import jax.numpy as jnp
from jax.experimental import pallas as pl
from jax.experimental.pallas import tpu as pltpu

_MIN_MASK_AREA = 120000
_THRESHOLD = 0.5
_N, _H, _W = 128, 512, 512


def _body(masks_ref, labels_ref, boxes_ref, ids_ref,
          masks_out_ref, labels_out_ref, boxes_out_ref, ids_out_ref,
          keep0_ref, stage_ref, idx_ref, cnt_ref, issued_ref, sem):
    i = pl.program_id(0)

    @pl.when(i == 0)
    def _():
        cnt_ref[0] = 0
        issued_ref[0] = 0
        keep0_ref[...] = masks_ref[...]

    area = jnp.sum((masks_ref[...] > _THRESHOLD).astype(jnp.float32))
    passes = area >= jnp.float32(_MIN_MASK_AREA)

    @pl.when(issued_ref[0] == 1)
    def _():
        pltpu.make_async_copy(
            stage_ref, masks_out_ref.at[pl.ds(0, 1)], sem).wait()
        issued_ref[0] = 0

    @pl.when(passes)
    def _():
        c = cnt_ref[0]
        idx_ref[c] = i
        stage_ref[...] = masks_ref[...]
        pltpu.make_async_copy(
            stage_ref, masks_out_ref.at[pl.ds(c, 1)], sem).start()
        issued_ref[0] = 1
        cnt_ref[0] = c + 1

    @pl.when(i == _N - 1)
    def _():
        @pl.when(issued_ref[0] == 1)
        def _():
            pltpu.make_async_copy(
                stage_ref, masks_out_ref.at[pl.ds(0, 1)], sem).wait()
            issued_ref[0] = 0

        c = cnt_ref[0]

        def _fill(k, carry):
            @pl.when(k >= c)
            def _():
                idx_ref[k] = 0
                cp = pltpu.make_async_copy(
                    keep0_ref, masks_out_ref.at[pl.ds(k, 1)], sem)
                cp.start()
                cp.wait()
            return carry

        jax.lax.fori_loop(0, _N, _fill, 0)

        def _gather(j, carry):
            t = idx_ref[j]
            labels_out_ref[j] = labels_ref[t]
            ids_out_ref[j] = ids_ref[t]
            for col in range(9):
                boxes_out_ref[j, col] = boxes_ref[t, col]
            return carry

        jax.lax.fori_loop(0, _N, _gather, 0)


def kernel(masks, labels, boxes_3d, instance_ids):
    out_shape = (
        jax.ShapeDtypeStruct((_N, _H, _W), jnp.float32),
        jax.ShapeDtypeStruct((_N,), jnp.int32),
        jax.ShapeDtypeStruct((_N, 9), jnp.float32),
        jax.ShapeDtypeStruct((_N,), jnp.int32),
    )
    return pl.pallas_call(
        _body,
        grid=(_N,),
        in_specs=[
            pl.BlockSpec((1, _H, _W), lambda i: (i, 0, 0)),
            pl.BlockSpec(memory_space=pltpu.SMEM),
            pl.BlockSpec(memory_space=pltpu.SMEM),
            pl.BlockSpec(memory_space=pltpu.SMEM),
        ],
        out_specs=(
            pl.BlockSpec(memory_space=pltpu.HBM),
            pl.BlockSpec(memory_space=pltpu.SMEM),
            pl.BlockSpec(memory_space=pltpu.SMEM),
            pl.BlockSpec(memory_space=pltpu.SMEM),
        ),
        out_shape=out_shape,
        scratch_shapes=[
            pltpu.VMEM((1, _H, _W), jnp.float32),
            pltpu.VMEM((1, _H, _W), jnp.float32),
            pltpu.SMEM((_N,), jnp.int32),
            pltpu.SMEM((1,), jnp.int32),
            pltpu.SMEM((1,), jnp.int32),
            pltpu.SemaphoreType.DMA,
        ],
    )(masks, labels, boxes_3d, instance_ids)


# speedup vs baseline: 2.0465x; 1.5845x over previous
import jax
import jax.numpy as jnp
from jax.experimental import pallas as pl
from jax.experimental.pallas import tpu as pltpu

_MIN_MASK_AREA = 120000
_THRESHOLD = 0.5
_N, _H, _W = 128, 512, 512
_RING = 6
_LOOK = 3


def _body(masks_ref, labels_ref, boxes_ref, ids_ref,
          masks_out_ref, labels_out_ref, boxes_out_ref, ids_out_ref,
          ring_ref, idx_ref, cnt_ref, flag_ref, in_sems, out_sems):
    i = pl.program_id(0)

    def _in_copy(j):
        s = j % _RING
        return pltpu.make_async_copy(
            masks_ref.at[pl.ds(j, 1)], ring_ref.at[pl.ds(s, 1)],
            in_sems.at[s])

    def _out_drain(s):
        pltpu.make_async_copy(
            ring_ref.at[pl.ds(s, 1)], masks_out_ref.at[pl.ds(0, 1)],
            out_sems.at[s]).wait()

    @pl.when(i == 0)
    def _():
        cnt_ref[0] = 0
        for s in range(_RING):
            flag_ref[s] = 0
        for j in range(_LOOK):
            _in_copy(j).start()

    @pl.when(i + _LOOK < _N)
    def _():
        s = (i + _LOOK) % _RING

        @pl.when(flag_ref[s] == 1)
        def _():
            _out_drain(s)
            flag_ref[s] = 0

        _in_copy(i + _LOOK).start()

    _in_copy(i).wait()
    s_i = i % _RING
    blk = ring_ref[pl.ds(s_i, 1)]
    sel = (blk.reshape(_H, _W) > _THRESHOLD).astype(jnp.float32)
    col = jax.lax.dot_general(
        jnp.ones((8, _H), jnp.float32), sel, (((1,), (0,)), ((), ())),
        preferred_element_type=jnp.float32)
    area = jnp.sum(col[0])

    @pl.when(area >= jnp.float32(_MIN_MASK_AREA))
    def _():
        c = cnt_ref[0]
        idx_ref[c] = i
        pltpu.make_async_copy(
            ring_ref.at[pl.ds(s_i, 1)], masks_out_ref.at[pl.ds(c, 1)],
            out_sems.at[s_i]).start()
        flag_ref[s_i] = 1
        cnt_ref[0] = c + 1

    @pl.when(i == _N - 1)
    def _():
        def _drain(s, carry):
            @pl.when(flag_ref[s] == 1)
            def _():
                _out_drain(s)
                flag_ref[s] = 0
            return carry

        jax.lax.fori_loop(0, _RING, _drain, 0)

        c = cnt_ref[0]

        def _fill(k, carry):
            @pl.when(k >= c)
            def _():
                idx_ref[k] = 0
                cp = pltpu.make_async_copy(
                    masks_ref.at[pl.ds(0, 1)],
                    masks_out_ref.at[pl.ds(k, 1)], out_sems.at[0])
                cp.start()
                cp.wait()
            return carry

        jax.lax.fori_loop(0, _N, _fill, 0)

        def _gather(j, carry):
            t = idx_ref[j]
            labels_out_ref[j] = labels_ref[t]
            ids_out_ref[j] = ids_ref[t]
            for col in range(9):
                boxes_out_ref[j, col] = boxes_ref[t, col]
            return carry

        jax.lax.fori_loop(0, _N, _gather, 0)


def kernel(masks, labels, boxes_3d, instance_ids):
    out_shape = (
        jax.ShapeDtypeStruct((_N, _H, _W), jnp.float32),
        jax.ShapeDtypeStruct((_N,), jnp.int32),
        jax.ShapeDtypeStruct((_N, 9), jnp.float32),
        jax.ShapeDtypeStruct((_N,), jnp.int32),
    )
    return pl.pallas_call(
        _body,
        grid=(_N,),
        in_specs=[
            pl.BlockSpec(memory_space=pltpu.HBM),
            pl.BlockSpec(memory_space=pltpu.SMEM),
            pl.BlockSpec(memory_space=pltpu.SMEM),
            pl.BlockSpec(memory_space=pltpu.SMEM),
        ],
        out_specs=(
            pl.BlockSpec(memory_space=pltpu.HBM),
            pl.BlockSpec(memory_space=pltpu.SMEM),
            pl.BlockSpec(memory_space=pltpu.SMEM),
            pl.BlockSpec(memory_space=pltpu.SMEM),
        ),
        out_shape=out_shape,
        scratch_shapes=[
            pltpu.VMEM((_RING, _H, _W), jnp.float32),
            pltpu.SMEM((_N,), jnp.int32),
            pltpu.SMEM((1,), jnp.int32),
            pltpu.SMEM((_RING,), jnp.int32),
            pltpu.SemaphoreType.DMA((_RING,)),
            pltpu.SemaphoreType.DMA((_RING,)),
        ],
    )(masks, labels, boxes_3d, instance_ids)


# speedup vs baseline: 2.1337x; 1.0426x over previous
import jax
import jax.numpy as jnp
from jax.experimental import pallas as pl
from jax.experimental.pallas import tpu as pltpu

_MIN_MASK_AREA = 120000
_THRESHOLD = 0.5
_N, _H, _W = 128, 512, 512
_G = 2
_NP = _N // _G
_RING = 5
_LOOK = 3


def _body(masks_ref, labels_ref, boxes_ref, ids_ref,
          masks_out_ref, labels_out_ref, boxes_out_ref, ids_out_ref,
          ring_ref, idx_ref, cnt_ref, flag_ref, in_sems, out_sems):
    j = pl.program_id(0)

    def _in_copy(p):
        s = p % _RING
        return pltpu.make_async_copy(
            masks_ref.at[pl.ds(p * _G, _G)],
            ring_ref.at[pl.ds(s * _G, _G)], in_sems.at[s])

    def _out_drain(s):
        @pl.when(flag_ref[s] == 1)
        def _():
            pltpu.make_async_copy(
                ring_ref.at[pl.ds(s * _G, 1)],
                masks_out_ref.at[pl.ds(0, 1)], out_sems.at[s]).wait()

        @pl.when(flag_ref[s] == 2)
        def _():
            pltpu.make_async_copy(
                ring_ref.at[pl.ds(s * _G, 2)],
                masks_out_ref.at[pl.ds(0, 2)], out_sems.at[s]).wait()

        flag_ref[s] = 0

    @pl.when(j == 0)
    def _():
        cnt_ref[0] = 0
        for s in range(_RING):
            flag_ref[s] = 0
        for p in range(_LOOK):
            _in_copy(p).start()

    @pl.when(j + _LOOK < _NP)
    def _():
        s = (j + _LOOK) % _RING

        @pl.when(flag_ref[s] > 0)
        def _():
            _out_drain(s)

        _in_copy(j + _LOOK).start()

    _in_copy(j).wait()
    s_j = j % _RING
    blk = ring_ref[pl.ds(s_j * _G, _G)]
    sel = (blk.reshape(_G * _H, _W) > _THRESHOLD).astype(jnp.float32)
    row = jax.lax.broadcasted_iota(jnp.int32, (8, _G * _H), 0)
    col = jax.lax.broadcasted_iota(jnp.int32, (8, _G * _H), 1)
    lhs = ((col // _H) == row).astype(jnp.float32)
    acc = jax.lax.dot_general(
        lhs, sel, (((1,), (0,)), ((), ())),
        preferred_element_type=jnp.float32)
    area0 = jnp.sum(acc[0])
    area1 = jnp.sum(acc[1])
    p0 = area0 >= jnp.float32(_MIN_MASK_AREA)
    p1 = area1 >= jnp.float32(_MIN_MASK_AREA)

    @pl.when(p0 & p1)
    def _():
        c = cnt_ref[0]
        idx_ref[c] = _G * j
        idx_ref[c + 1] = _G * j + 1
        pltpu.make_async_copy(
            ring_ref.at[pl.ds(s_j * _G, 2)],
            masks_out_ref.at[pl.ds(c, 2)], out_sems.at[s_j]).start()
        flag_ref[s_j] = 2
        cnt_ref[0] = c + 2

    @pl.when(p0 & jnp.logical_not(p1))
    def _():
        c = cnt_ref[0]
        idx_ref[c] = _G * j
        pltpu.make_async_copy(
            ring_ref.at[pl.ds(s_j * _G, 1)],
            masks_out_ref.at[pl.ds(c, 1)], out_sems.at[s_j]).start()
        flag_ref[s_j] = 1
        cnt_ref[0] = c + 1

    @pl.when(jnp.logical_not(p0) & p1)
    def _():
        c = cnt_ref[0]
        idx_ref[c] = _G * j + 1
        pltpu.make_async_copy(
            ring_ref.at[pl.ds(s_j * _G + 1, 1)],
            masks_out_ref.at[pl.ds(c, 1)], out_sems.at[s_j]).start()
        flag_ref[s_j] = 1
        cnt_ref[0] = c + 1

    @pl.when(j == _NP - 1)
    def _():
        def _drain(s, carry):
            @pl.when(flag_ref[s] > 0)
            def _():
                _out_drain(s)
            return carry

        jax.lax.fori_loop(0, _RING, _drain, 0)

        c = cnt_ref[0]

        def _fill(k, carry):
            @pl.when(k >= c)
            def _():
                idx_ref[k] = 0
                cp = pltpu.make_async_copy(
                    masks_ref.at[pl.ds(0, 1)],
                    masks_out_ref.at[pl.ds(k, 1)], out_sems.at[0])
                cp.start()
                cp.wait()
            return carry

        jax.lax.fori_loop(0, _N, _fill, 0)

        def _gather(q, carry):
            t = idx_ref[q]
            labels_out_ref[q] = labels_ref[t]
            ids_out_ref[q] = ids_ref[t]
            for colk in range(9):
                boxes_out_ref[q, colk] = boxes_ref[t, colk]
            return carry

        jax.lax.fori_loop(0, _N, _gather, 0)


def kernel(masks, labels, boxes_3d, instance_ids):
    out_shape = (
        jax.ShapeDtypeStruct((_N, _H, _W), jnp.float32),
        jax.ShapeDtypeStruct((_N,), jnp.int32),
        jax.ShapeDtypeStruct((_N, 9), jnp.float32),
        jax.ShapeDtypeStruct((_N,), jnp.int32),
    )
    return pl.pallas_call(
        _body,
        grid=(_NP,),
        in_specs=[
            pl.BlockSpec(memory_space=pltpu.HBM),
            pl.BlockSpec(memory_space=pltpu.SMEM),
            pl.BlockSpec(memory_space=pltpu.SMEM),
            pl.BlockSpec(memory_space=pltpu.SMEM),
        ],
        out_specs=(
            pl.BlockSpec(memory_space=pltpu.HBM),
            pl.BlockSpec(memory_space=pltpu.SMEM),
            pl.BlockSpec(memory_space=pltpu.SMEM),
            pl.BlockSpec(memory_space=pltpu.SMEM),
        ),
        out_shape=out_shape,
        scratch_shapes=[
            pltpu.VMEM((_RING * _G, _H, _W), jnp.float32),
            pltpu.SMEM((_N,), jnp.int32),
            pltpu.SMEM((1,), jnp.int32),
            pltpu.SMEM((_RING,), jnp.int32),
            pltpu.SemaphoreType.DMA((_RING,)),
            pltpu.SemaphoreType.DMA((_RING,)),
        ],
    )(masks, labels, boxes_3d, instance_ids)


# speedup vs baseline: 2.1745x; 1.0191x over previous
import jax
import jax.numpy as jnp
from jax.experimental import pallas as pl
from jax.experimental.pallas import tpu as pltpu

_MIN_MASK_AREA = 120000
_THRESHOLD = 0.5
_N, _H, _W = 128, 512, 512
_G = 4
_NP = _N // _G
_RING = 4
_LOOK = 2


def _body(masks_ref, labels_ref, boxes_ref, ids_ref,
          masks_out_ref, labels_out_ref, boxes_out_ref, ids_out_ref,
          ring_ref, idx_ref, cnt_ref, flag_ref, in_sems, out_sems):
    j = pl.program_id(0)

    def _in_copy(p):
        s = p % _RING
        return pltpu.make_async_copy(
            masks_ref.at[pl.ds(p * _G, _G)],
            ring_ref.at[pl.ds(s * _G, _G)], in_sems.at[s])

    def _out_drain(s):
        def _w(u, carry):
            pltpu.make_async_copy(
                ring_ref.at[pl.ds(s * _G, 1)],
                masks_out_ref.at[pl.ds(0, 1)], out_sems.at[s]).wait()
            return carry

        jax.lax.fori_loop(0, flag_ref[s], _w, 0)
        flag_ref[s] = 0

    @pl.when(j == 0)
    def _():
        cnt_ref[0] = 0
        for s in range(_RING):
            flag_ref[s] = 0
        for p in range(_LOOK):
            _in_copy(p).start()

    @pl.when(j + _LOOK < _NP)
    def _():
        s = (j + _LOOK) % _RING

        @pl.when(flag_ref[s] > 0)
        def _():
            _out_drain(s)

        _in_copy(j + _LOOK).start()

    _in_copy(j).wait()
    s_j = j % _RING
    blk = ring_ref[pl.ds(s_j * _G, _G)]
    sel = (blk.reshape(_G * _H, _W) > _THRESHOLD).astype(jnp.float32)
    row = jax.lax.broadcasted_iota(jnp.int32, (8, _G * _H), 0)
    col = jax.lax.broadcasted_iota(jnp.int32, (8, _G * _H), 1)
    lhs = ((col // _H) == row).astype(jnp.float32)
    acc = jax.lax.dot_general(
        lhs, sel, (((1,), (0,)), ((), ())),
        preferred_element_type=jnp.float32)
    passes = [jnp.sum(acc[k]) >= jnp.float32(_MIN_MASK_AREA)
              for k in range(_G)]
    all_pass = passes[0] & passes[1] & passes[2] & passes[3]

    @pl.when(all_pass)
    def _():
        c = cnt_ref[0]
        for k in range(_G):
            idx_ref[c + k] = _G * j + k
        pltpu.make_async_copy(
            ring_ref.at[pl.ds(s_j * _G, _G)],
            masks_out_ref.at[pl.ds(c, _G)], out_sems.at[s_j]).start()
        flag_ref[s_j] = _G
        cnt_ref[0] = c + _G

    @pl.when(jnp.logical_not(all_pass))
    def _():
        for k in range(_G):
            @pl.when(passes[k])
            def _(k=k):
                c = cnt_ref[0]
                idx_ref[c] = _G * j + k
                pltpu.make_async_copy(
                    ring_ref.at[pl.ds(s_j * _G + k, 1)],
                    masks_out_ref.at[pl.ds(c, 1)], out_sems.at[s_j]).start()
                flag_ref[s_j] = flag_ref[s_j] + 1
                cnt_ref[0] = c + 1

    @pl.when(j == _NP - 1)
    def _():
        def _drain(s, carry):
            @pl.when(flag_ref[s] > 0)
            def _():
                _out_drain(s)
            return carry

        jax.lax.fori_loop(0, _RING, _drain, 0)

        c = cnt_ref[0]

        def _fill(k, carry):
            @pl.when(k >= c)
            def _():
                idx_ref[k] = 0
                cp = pltpu.make_async_copy(
                    masks_ref.at[pl.ds(0, 1)],
                    masks_out_ref.at[pl.ds(k, 1)], out_sems.at[0])
                cp.start()
                cp.wait()
            return carry

        jax.lax.fori_loop(0, _N, _fill, 0)

        def _gather(q, carry):
            t = idx_ref[q]
            labels_out_ref[q] = labels_ref[t]
            ids_out_ref[q] = ids_ref[t]
            for colk in range(9):
                boxes_out_ref[q, colk] = boxes_ref[t, colk]
            return carry

        jax.lax.fori_loop(0, _N, _gather, 0)


def kernel(masks, labels, boxes_3d, instance_ids):
    out_shape = (
        jax.ShapeDtypeStruct((_N, _H, _W), jnp.float32),
        jax.ShapeDtypeStruct((_N,), jnp.int32),
        jax.ShapeDtypeStruct((_N, 9), jnp.float32),
        jax.ShapeDtypeStruct((_N,), jnp.int32),
    )
    return pl.pallas_call(
        _body,
        grid=(_NP,),
        in_specs=[
            pl.BlockSpec(memory_space=pltpu.HBM),
            pl.BlockSpec(memory_space=pltpu.SMEM),
            pl.BlockSpec(memory_space=pltpu.SMEM),
            pl.BlockSpec(memory_space=pltpu.SMEM),
        ],
        out_specs=(
            pl.BlockSpec(memory_space=pltpu.HBM),
            pl.BlockSpec(memory_space=pltpu.SMEM),
            pl.BlockSpec(memory_space=pltpu.SMEM),
            pl.BlockSpec(memory_space=pltpu.SMEM),
        ),
        out_shape=out_shape,
        scratch_shapes=[
            pltpu.VMEM((_RING * _G, _H, _W), jnp.float32),
            pltpu.SMEM((_N,), jnp.int32),
            pltpu.SMEM((1,), jnp.int32),
            pltpu.SMEM((_RING,), jnp.int32),
            pltpu.SemaphoreType.DMA((_RING,)),
            pltpu.SemaphoreType.DMA((_RING,)),
        ],
    )(masks, labels, boxes_3d, instance_ids)
